# Initial kernel scaffold; baseline (speedup 1.0000x reference)
import jax
import jax.numpy as jnp
from jax.experimental import pallas as pl


def kernel(x, edge_index, steps, enc_W1, enc_b1, enc_W2, enc_b2, mp_W, mp_b, dec_W1, dec_b1, dec_W2, dec_b2):
    raise NotImplementedError("write your pallas kernel here")


# speedup vs baseline: 3.9050x; 3.9050x over previous
import functools

import jax
import jax.numpy as jnp
from jax import lax
from jax.experimental import pallas as pl
from jax.experimental.pallas import tpu as pltpu
from jax.experimental.pallas import tpu_sc as plsc

_CHUNK = 64
_NCORES = 2
_NSUB = 16
_NW = _NCORES * _NSUB
_DEGW = 16
_IDXG = 8
_HH = 64


def _plan(n, n_streams):
    n_pad = ((n + 8 * _NSUB - 1) // (8 * _NSUB)) * (8 * _NSUB)
    rows_per_tile = n_pad // _NSUB
    cmax = ((-(-n_streams // _NW) + 7) // 8) * 8
    q64, rem = divmod(rows_per_tile, _CHUNK)
    return n_pad, rows_per_tile, cmax, q64, rem


def _seg_sum_half(h_half, src2d, dst2d, n_streams):
    n = h_half.shape[0]
    n_pad, rows_per_tile, cmax, q64, rem = _plan(n, n_streams)
    zrow = jnp.zeros((_CHUNK, _HH), jnp.float32)

    mesh = plsc.VectorSubcoreMesh(core_axis_name="c", subcore_axis_name="s")

    @functools.partial(
        pl.kernel,
        mesh=mesh,
        out_type=jax.ShapeDtypeStruct((_NCORES, n_pad, _HH), jnp.float32),
        scratch_types=[
            pltpu.VMEM((_IDXG, _CHUNK), jnp.int32),
            pltpu.VMEM((_IDXG, _CHUNK), jnp.int32),
            pltpu.VMEM((2, _CHUNK, _HH), jnp.float32),
            pltpu.VMEM_SHARED((n_pad, _HH), jnp.float32),
            pltpu.SemaphoreType.DMA,
        ],
        compiler_params=pltpu.CompilerParams(use_tc_tiling_on_sc=False),
    )
    def seg_kernel(h_hbm, src_hbm, dst_hbm, zrow_hbm, parts_hbm,
                   srcg, dstg, rows, acc, g0):
        cid = lax.axis_index("c")
        sid = lax.axis_index("s")
        w = cid * _NSUB + sid
        base = w * cmax
        cnt = jnp.clip(n_streams - base, 0, cmax)
        row0 = sid * rows_per_tile

        pltpu.sync_copy(zrow_hbm, rows.at[0])
        for j in range(q64):
            pltpu.sync_copy(rows.at[0],
                            acc.at[pl.ds(row0 + j * _CHUNK, _CHUNK)])
        if rem:
            pltpu.sync_copy(rows.at[0, pl.ds(0, rem)],
                            acc.at[pl.ds(row0 + q64 * _CHUNK, rem)])
        plsc.subcore_barrier()

        def body(k, carry):
            @pl.when((k % _IDXG) == 0)
            def _():
                g = k // _IDXG
                pltpu.sync_copy(
                    src_hbm.at[pl.ds(base + g * _IDXG, _IDXG)], srcg)
                pltpu.sync_copy(
                    dst_hbm.at[pl.ds(base + g * _IDXG, _IDXG)], dstg)

            ks = k % _IDXG
            pltpu.async_copy(h_hbm.at[srcg.at[ks]], rows.at[0], g0).wait()
            pltpu.sync_copy(rows.at[0], acc.at[dstg.at[ks]], add=True)
            return carry

        lax.fori_loop(0, cnt, body, 0)

        plsc.subcore_barrier()
        for j in range(q64):
            pltpu.sync_copy(acc.at[pl.ds(row0 + j * _CHUNK, _CHUNK)],
                            rows.at[0])
            pltpu.sync_copy(rows.at[0],
                            parts_hbm.at[cid, pl.ds(row0 + j * _CHUNK,
                                                    _CHUNK)])
        if rem:
            pltpu.sync_copy(acc.at[pl.ds(row0 + q64 * _CHUNK, rem)],
                            rows.at[0, pl.ds(0, rem)])
            pltpu.sync_copy(rows.at[0, pl.ds(0, rem)],
                            parts_hbm.at[cid, pl.ds(row0 + q64 * _CHUNK,
                                                    rem)])

    return seg_kernel(h_half, src2d, dst2d, zrow)


def _deg_sc(dst2d, n, n_streams):
    n_pad, rows_per_tile, cmax, q64, rem = _plan(n, n_streams)
    zdeg = jnp.zeros((_CHUNK, _DEGW), jnp.float32)
    ones16 = jnp.ones((_CHUNK, _DEGW), jnp.float32)

    mesh = plsc.VectorSubcoreMesh(core_axis_name="c", subcore_axis_name="s")

    @functools.partial(
        pl.kernel,
        mesh=mesh,
        out_type=jax.ShapeDtypeStruct((_NCORES, n_pad, _DEGW), jnp.float32),
        scratch_types=[
            pltpu.VMEM((_IDXG, _CHUNK), jnp.int32),
            pltpu.VMEM((_CHUNK, _DEGW), jnp.float32),
            pltpu.VMEM((_CHUNK, _DEGW), jnp.float32),
            pltpu.VMEM_SHARED((n_pad, _DEGW), jnp.float32),
        ],
        compiler_params=pltpu.CompilerParams(use_tc_tiling_on_sc=False),
    )
    def deg_kernel(dst_hbm, zdeg_hbm, ones_hbm, degp_hbm,
                   dstg, ones_v, deg_v, deg_sh):
        cid = lax.axis_index("c")
        sid = lax.axis_index("s")
        w = cid * _NSUB + sid
        base = w * cmax
        cnt = jnp.clip(n_streams - base, 0, cmax)
        row0 = sid * rows_per_tile

        pltpu.sync_copy(zdeg_hbm, deg_v)
        for j in range(q64):
            pltpu.sync_copy(deg_v,
                            deg_sh.at[pl.ds(row0 + j * _CHUNK, _CHUNK)])
        if rem:
            pltpu.sync_copy(deg_v.at[pl.ds(0, rem)],
                            deg_sh.at[pl.ds(row0 + q64 * _CHUNK, rem)])
        pltpu.sync_copy(ones_hbm, ones_v)
        plsc.subcore_barrier()

        def body(k, carry):
            @pl.when((k % _IDXG) == 0)
            def _():
                g = k // _IDXG
                pltpu.sync_copy(
                    dst_hbm.at[pl.ds(base + g * _IDXG, _IDXG)], dstg)

            pltpu.sync_copy(ones_v, deg_sh.at[dstg.at[k % _IDXG]], add=True)
            return carry

        lax.fori_loop(0, cnt, body, 0)

        plsc.subcore_barrier()
        for j in range(q64):
            pltpu.sync_copy(deg_sh.at[pl.ds(row0 + j * _CHUNK, _CHUNK)],
                            deg_v)
            pltpu.sync_copy(deg_v,
                            degp_hbm.at[cid, pl.ds(row0 + j * _CHUNK,
                                                   _CHUNK)])
        if rem:
            pltpu.sync_copy(deg_sh.at[pl.ds(row0 + q64 * _CHUNK, rem)],
                            deg_v.at[pl.ds(0, rem)])
            pltpu.sync_copy(deg_v.at[pl.ds(0, rem)],
                            degp_hbm.at[cid, pl.ds(row0 + q64 * _CHUNK,
                                                   rem)])

    return deg_kernel(dst2d, zdeg, ones16)


def _enc_body(x_ref, w1_ref, b1_ref, w2_ref, b2_ref, lo_ref, hi_ref):
    a = jnp.dot(x_ref[...], w1_ref[...], preferred_element_type=jnp.float32)
    a = jnp.maximum(a + b1_ref[...], 0.0)
    b = jnp.dot(a, w2_ref[...], preferred_element_type=jnp.float32)
    h = jnp.maximum(b + b2_ref[...], 0.0)
    lo_ref[...] = h[:, :_HH]
    hi_ref[...] = h[:, _HH:]


def _dec_body(plo_ref, phi_ref, degp_ref, hlo_ref, hhi_ref,
              mpwl_ref, mpwh_ref, mpb_ref, w1a_ref, w1bl_ref, w1bh_ref,
              b1_ref, w2_ref, b2_ref, o_ref):
    seg_lo = plo_ref[0] + plo_ref[1]
    seg_hi = phi_ref[0] + phi_ref[1]
    deg = degp_ref[0, :, 0:1] + degp_ref[1, :, 0:1]
    agg = (jnp.dot(seg_lo, mpwl_ref[...], preferred_element_type=jnp.float32)
           + jnp.dot(seg_hi, mpwh_ref[...],
                     preferred_element_type=jnp.float32)
           + deg * mpb_ref[...])
    u = (jnp.dot(agg, w1a_ref[...], preferred_element_type=jnp.float32)
         + jnp.dot(hlo_ref[...], w1bl_ref[...],
                   preferred_element_type=jnp.float32)
         + jnp.dot(hhi_ref[...], w1bh_ref[...],
                   preferred_element_type=jnp.float32)
         + b1_ref[...])
    u = jnp.maximum(u, 0.0)
    v = jnp.dot(u, w2_ref[...], preferred_element_type=jnp.float32)
    o_ref[...] = jnp.tanh(v + b2_ref[...])


def kernel(x, edge_index, steps, enc_W1, enc_b1, enc_W2, enc_b2, mp_W, mp_b,
           dec_W1, dec_b1, dec_W2, dec_b2):
    n, d_in = x.shape
    h_dim = mp_W.shape[0]
    mlp_h = enc_W1.shape[1]
    e = edge_index.shape[1]

    n_streams = e // _CHUNK
    cmax = ((-(-n_streams // _NW) + 7) // 8) * 8
    pad_streams = _NW * cmax

    src = edge_index[0]
    dst = edge_index[1]
    pad_e = pad_streams * _CHUNK - e
    src2d = jnp.pad(src, (0, pad_e)).reshape(pad_streams, _CHUNK)
    dst2d = jnp.pad(dst, (0, pad_e)).reshape(pad_streams, _CHUNK)

    b1e = enc_b1.reshape(1, mlp_h)
    b2e = enc_b2.reshape(1, h_dim)
    mpb = mp_b.reshape(1, h_dim)
    mpw_lo = mp_W[:_HH]
    mpw_hi = mp_W[_HH:]
    w1a = dec_W1[:h_dim]
    w1b_lo = dec_W1[h_dim:h_dim + _HH]
    w1b_hi = dec_W1[h_dim + _HH:]
    b1d = dec_b1.reshape(1, mlp_h)
    b2d = dec_b2.reshape(1, d_in)

    rb = 2000
    grid = (n // rb,)

    def full(shape):
        return pl.BlockSpec(shape, lambda i: tuple(0 for _ in shape))

    enc_call = pl.pallas_call(
        _enc_body,
        grid=grid,
        in_specs=[
            pl.BlockSpec((rb, d_in), lambda i: (i, 0)),
            full((d_in, mlp_h)),
            full((1, mlp_h)),
            full((mlp_h, h_dim)),
            full((1, h_dim)),
        ],
        out_specs=[pl.BlockSpec((rb, _HH), lambda i: (i, 0)),
                   pl.BlockSpec((rb, _HH), lambda i: (i, 0))],
        out_shape=[jax.ShapeDtypeStruct((n, _HH), jnp.float32),
                   jax.ShapeDtypeStruct((n, _HH), jnp.float32)],
    )

    dec_call = pl.pallas_call(
        _dec_body,
        grid=grid,
        in_specs=[
            pl.BlockSpec((_NCORES, rb, _HH), lambda i: (0, i, 0)),
            pl.BlockSpec((_NCORES, rb, _HH), lambda i: (0, i, 0)),
            pl.BlockSpec((_NCORES, rb, _DEGW), lambda i: (0, i, 0)),
            pl.BlockSpec((rb, _HH), lambda i: (i, 0)),
            pl.BlockSpec((rb, _HH), lambda i: (i, 0)),
            full((_HH, h_dim)),
            full((_HH, h_dim)),
            full((1, h_dim)),
            full((h_dim, mlp_h)),
            full((_HH, mlp_h)),
            full((_HH, mlp_h)),
            full((1, mlp_h)),
            full((mlp_h, d_in)),
            full((1, d_in)),
        ],
        out_specs=pl.BlockSpec((rb, d_in), lambda i: (i, 0)),
        out_shape=jax.ShapeDtypeStruct((n, d_in), jnp.float32),
    )

    degp = _deg_sc(dst2d, n, n_streams)[:, :n]

    def body(_, xc):
        h_lo, h_hi = enc_call(xc, enc_W1, b1e, enc_W2, b2e)
        p_lo = _seg_sum_half(h_lo, src2d, dst2d, n_streams)[:, :n]
        p_hi = _seg_sum_half(h_hi, src2d, dst2d, n_streams)[:, :n]
        return dec_call(p_lo, p_hi, degp, h_lo, h_hi,
                        mpw_lo, mpw_hi, mpb, w1a, w1b_lo, w1b_hi,
                        b1d, dec_W2, b2d)

    return lax.fori_loop(0, steps, body, x)


# speedup vs baseline: 4.7602x; 1.2190x over previous
import functools

import jax
import jax.numpy as jnp
from jax import lax
from jax.experimental import pallas as pl
from jax.experimental.pallas import tpu as pltpu
from jax.experimental.pallas import tpu_sc as plsc

_CHUNK = 64
_NCORES = 2
_NSUB = 16
_NW = _NCORES * _NSUB
_DEGW = 16
_IDXG = 8
_HH = 64


def _plan(n, n_streams):
    n_pad = ((n + 8 * _NSUB - 1) // (8 * _NSUB)) * (8 * _NSUB)
    rows_per_tile = n_pad // _NSUB
    cmax = ((-(-n_streams // _NW) + 7) // 8) * 8
    q64, rem = divmod(rows_per_tile, _CHUNK)
    return n_pad, rows_per_tile, cmax, q64, rem


def _seg_sum_half(h_half, src2d, dst2d, n_streams):
    n = h_half.shape[0]
    n_pad, rows_per_tile, cmax, q64, rem = _plan(n, n_streams)
    zrow = jnp.zeros((_CHUNK, _HH), jnp.float32)

    mesh = plsc.VectorSubcoreMesh(core_axis_name="c", subcore_axis_name="s")

    @functools.partial(
        pl.kernel,
        mesh=mesh,
        out_type=jax.ShapeDtypeStruct((_NCORES, n_pad, _HH), jnp.float32),
        scratch_types=[
            pltpu.VMEM((2, _IDXG, _CHUNK), jnp.int32),
            pltpu.VMEM((2, _IDXG, _CHUNK), jnp.int32),
            pltpu.VMEM((2, _CHUNK, _HH), jnp.float32),
            pltpu.VMEM_SHARED((n_pad, _HH), jnp.float32),
            pltpu.SemaphoreType.DMA,
            pltpu.SemaphoreType.DMA,
        ],
        compiler_params=pltpu.CompilerParams(use_tc_tiling_on_sc=False),
    )
    def seg_kernel(h_hbm, src_hbm, dst_hbm, zrow_hbm, parts_hbm,
                   srcg, dstg, rows, acc, g0, g1):
        cid = lax.axis_index("c")
        sid = lax.axis_index("s")
        w = cid * _NSUB + sid
        base = w * cmax
        cnt = jnp.clip(n_streams - base, 0, cmax)
        row0 = sid * rows_per_tile

        pltpu.sync_copy(zrow_hbm, rows.at[0])
        for j in range(q64):
            pltpu.sync_copy(rows.at[0],
                            acc.at[pl.ds(row0 + j * _CHUNK, _CHUNK)])
        if rem:
            pltpu.sync_copy(rows.at[0, pl.ds(0, rem)],
                            acc.at[pl.ds(row0 + q64 * _CHUNK, rem)])
        plsc.subcore_barrier()

        def load_idx_group(g):
            slot = g & 1
            pltpu.sync_copy(src_hbm.at[pl.ds(base + g * _IDXG, _IDXG)],
                            srcg.at[slot])
            pltpu.sync_copy(dst_hbm.at[pl.ds(base + g * _IDXG, _IDXG)],
                            dstg.at[slot])

        def src_at(k):
            return srcg.at[(k // _IDXG) & 1, k % _IDXG]

        def dst_at(k):
            return dstg.at[(k // _IDXG) & 1, k % _IDXG]

        def half_iter(k, b, gsem, gsem_n):
            pltpu.make_async_copy(h_hbm.at[src_at(k)], rows.at[b],
                                  gsem).wait()

            @pl.when(k + 1 < cnt)
            def _():
                pltpu.async_copy(h_hbm.at[src_at(k + 1)], rows.at[1 - b],
                                 gsem_n)

            pltpu.sync_copy(rows.at[b], acc.at[dst_at(k)], add=True)

        def body(k, carry):
            @pl.when((((k + 1) % _IDXG) == 0) & (k + 1 < cnt))
            def _():
                load_idx_group((k + 1) // _IDXG)

            @pl.when((k & 1) == 0)
            def _():
                half_iter(k, 0, g0, g1)

            @pl.when((k & 1) == 1)
            def _():
                half_iter(k, 1, g1, g0)

            return carry

        @pl.when(cnt > 0)
        def _():
            load_idx_group(0)
            pltpu.async_copy(h_hbm.at[src_at(0)], rows.at[0], g0)
            lax.fori_loop(0, cnt, body, 0)

        plsc.subcore_barrier()
        for j in range(q64):
            pltpu.sync_copy(acc.at[pl.ds(row0 + j * _CHUNK, _CHUNK)],
                            rows.at[0])
            pltpu.sync_copy(rows.at[0],
                            parts_hbm.at[cid, pl.ds(row0 + j * _CHUNK,
                                                    _CHUNK)])
        if rem:
            pltpu.sync_copy(acc.at[pl.ds(row0 + q64 * _CHUNK, rem)],
                            rows.at[0, pl.ds(0, rem)])
            pltpu.sync_copy(rows.at[0, pl.ds(0, rem)],
                            parts_hbm.at[cid, pl.ds(row0 + q64 * _CHUNK,
                                                    rem)])

    return seg_kernel(h_half, src2d, dst2d, zrow)


def _deg_sc(dst2d, n, n_streams):
    n_pad, rows_per_tile, cmax, q64, rem = _plan(n, n_streams)
    zdeg = jnp.zeros((_CHUNK, _DEGW), jnp.float32)
    ones16 = jnp.ones((_CHUNK, _DEGW), jnp.float32)

    mesh = plsc.VectorSubcoreMesh(core_axis_name="c", subcore_axis_name="s")

    @functools.partial(
        pl.kernel,
        mesh=mesh,
        out_type=jax.ShapeDtypeStruct((_NCORES, n_pad, _DEGW), jnp.float32),
        scratch_types=[
            pltpu.VMEM((_IDXG, _CHUNK), jnp.int32),
            pltpu.VMEM((_CHUNK, _DEGW), jnp.float32),
            pltpu.VMEM((_CHUNK, _DEGW), jnp.float32),
            pltpu.VMEM_SHARED((n_pad, _DEGW), jnp.float32),
        ],
        compiler_params=pltpu.CompilerParams(use_tc_tiling_on_sc=False),
    )
    def deg_kernel(dst_hbm, zdeg_hbm, ones_hbm, degp_hbm,
                   dstg, ones_v, deg_v, deg_sh):
        cid = lax.axis_index("c")
        sid = lax.axis_index("s")
        w = cid * _NSUB + sid
        base = w * cmax
        cnt = jnp.clip(n_streams - base, 0, cmax)
        row0 = sid * rows_per_tile

        pltpu.sync_copy(zdeg_hbm, deg_v)
        for j in range(q64):
            pltpu.sync_copy(deg_v,
                            deg_sh.at[pl.ds(row0 + j * _CHUNK, _CHUNK)])
        if rem:
            pltpu.sync_copy(deg_v.at[pl.ds(0, rem)],
                            deg_sh.at[pl.ds(row0 + q64 * _CHUNK, rem)])
        pltpu.sync_copy(ones_hbm, ones_v)
        plsc.subcore_barrier()

        def body(k, carry):
            @pl.when((k % _IDXG) == 0)
            def _():
                g = k // _IDXG
                pltpu.sync_copy(
                    dst_hbm.at[pl.ds(base + g * _IDXG, _IDXG)], dstg)

            pltpu.sync_copy(ones_v, deg_sh.at[dstg.at[k % _IDXG]], add=True)
            return carry

        lax.fori_loop(0, cnt, body, 0)

        plsc.subcore_barrier()
        for j in range(q64):
            pltpu.sync_copy(deg_sh.at[pl.ds(row0 + j * _CHUNK, _CHUNK)],
                            deg_v)
            pltpu.sync_copy(deg_v,
                            degp_hbm.at[cid, pl.ds(row0 + j * _CHUNK,
                                                   _CHUNK)])
        if rem:
            pltpu.sync_copy(deg_sh.at[pl.ds(row0 + q64 * _CHUNK, rem)],
                            deg_v.at[pl.ds(0, rem)])
            pltpu.sync_copy(deg_v.at[pl.ds(0, rem)],
                            degp_hbm.at[cid, pl.ds(row0 + q64 * _CHUNK,
                                                   rem)])

    return deg_kernel(dst2d, zdeg, ones16)


def _enc_body(x_ref, w1_ref, b1_ref, w2_ref, b2_ref, lo_ref, hi_ref):
    a = jnp.dot(x_ref[...], w1_ref[...], preferred_element_type=jnp.float32)
    a = jnp.maximum(a + b1_ref[...], 0.0)
    b = jnp.dot(a, w2_ref[...], preferred_element_type=jnp.float32)
    h = jnp.maximum(b + b2_ref[...], 0.0)
    lo_ref[...] = h[:, :_HH]
    hi_ref[...] = h[:, _HH:]


def _dec_body(plo_ref, phi_ref, degp_ref, hlo_ref, hhi_ref,
              mpwl_ref, mpwh_ref, mpb_ref, w1a_ref, w1bl_ref, w1bh_ref,
              b1_ref, w2_ref, b2_ref, o_ref):
    seg_lo = plo_ref[0] + plo_ref[1]
    seg_hi = phi_ref[0] + phi_ref[1]
    deg = degp_ref[0, :, 0:1] + degp_ref[1, :, 0:1]
    agg = (jnp.dot(seg_lo, mpwl_ref[...], preferred_element_type=jnp.float32)
           + jnp.dot(seg_hi, mpwh_ref[...],
                     preferred_element_type=jnp.float32)
           + deg * mpb_ref[...])
    u = (jnp.dot(agg, w1a_ref[...], preferred_element_type=jnp.float32)
         + jnp.dot(hlo_ref[...], w1bl_ref[...],
                   preferred_element_type=jnp.float32)
         + jnp.dot(hhi_ref[...], w1bh_ref[...],
                   preferred_element_type=jnp.float32)
         + b1_ref[...])
    u = jnp.maximum(u, 0.0)
    v = jnp.dot(u, w2_ref[...], preferred_element_type=jnp.float32)
    o_ref[...] = jnp.tanh(v + b2_ref[...])


def kernel(x, edge_index, steps, enc_W1, enc_b1, enc_W2, enc_b2, mp_W, mp_b,
           dec_W1, dec_b1, dec_W2, dec_b2):
    n, d_in = x.shape
    h_dim = mp_W.shape[0]
    mlp_h = enc_W1.shape[1]
    e = edge_index.shape[1]

    n_streams = e // _CHUNK
    cmax = ((-(-n_streams // _NW) + 7) // 8) * 8
    pad_streams = _NW * cmax

    src = edge_index[0]
    dst = edge_index[1]
    pad_e = pad_streams * _CHUNK - e
    src2d = jnp.pad(src, (0, pad_e)).reshape(pad_streams, _CHUNK)
    dst2d = jnp.pad(dst, (0, pad_e)).reshape(pad_streams, _CHUNK)

    b1e = enc_b1.reshape(1, mlp_h)
    b2e = enc_b2.reshape(1, h_dim)
    mpb = mp_b.reshape(1, h_dim)
    mpw_lo = mp_W[:_HH]
    mpw_hi = mp_W[_HH:]
    w1a = dec_W1[:h_dim]
    w1b_lo = dec_W1[h_dim:h_dim + _HH]
    w1b_hi = dec_W1[h_dim + _HH:]
    b1d = dec_b1.reshape(1, mlp_h)
    b2d = dec_b2.reshape(1, d_in)

    rb = 2000
    grid = (n // rb,)

    def full(shape):
        return pl.BlockSpec(shape, lambda i: tuple(0 for _ in shape))

    enc_call = pl.pallas_call(
        _enc_body,
        grid=grid,
        in_specs=[
            pl.BlockSpec((rb, d_in), lambda i: (i, 0)),
            full((d_in, mlp_h)),
            full((1, mlp_h)),
            full((mlp_h, h_dim)),
            full((1, h_dim)),
        ],
        out_specs=[pl.BlockSpec((rb, _HH), lambda i: (i, 0)),
                   pl.BlockSpec((rb, _HH), lambda i: (i, 0))],
        out_shape=[jax.ShapeDtypeStruct((n, _HH), jnp.float32),
                   jax.ShapeDtypeStruct((n, _HH), jnp.float32)],
    )

    dec_call = pl.pallas_call(
        _dec_body,
        grid=grid,
        in_specs=[
            pl.BlockSpec((_NCORES, rb, _HH), lambda i: (0, i, 0)),
            pl.BlockSpec((_NCORES, rb, _HH), lambda i: (0, i, 0)),
            pl.BlockSpec((_NCORES, rb, _DEGW), lambda i: (0, i, 0)),
            pl.BlockSpec((rb, _HH), lambda i: (i, 0)),
            pl.BlockSpec((rb, _HH), lambda i: (i, 0)),
            full((_HH, h_dim)),
            full((_HH, h_dim)),
            full((1, h_dim)),
            full((h_dim, mlp_h)),
            full((_HH, mlp_h)),
            full((_HH, mlp_h)),
            full((1, mlp_h)),
            full((mlp_h, d_in)),
            full((1, d_in)),
        ],
        out_specs=pl.BlockSpec((rb, d_in), lambda i: (i, 0)),
        out_shape=jax.ShapeDtypeStruct((n, d_in), jnp.float32),
    )

    degp = _deg_sc(dst2d, n, n_streams)[:, :n]

    def body(_, xc):
        h_lo, h_hi = enc_call(xc, enc_W1, b1e, enc_W2, b2e)
        p_lo = _seg_sum_half(h_lo, src2d, dst2d, n_streams)[:, :n]
        p_hi = _seg_sum_half(h_hi, src2d, dst2d, n_streams)[:, :n]
        return dec_call(p_lo, p_hi, degp, h_lo, h_hi,
                        mpw_lo, mpw_hi, mpb, w1a, w1b_lo, w1b_hi,
                        b1d, dec_W2, b2d)

    return lax.fori_loop(0, steps, body, x)


# speedup vs baseline: 4.8319x; 1.0151x over previous
import functools

import jax
import jax.numpy as jnp
from jax import lax
from jax.experimental import pallas as pl
from jax.experimental.pallas import tpu as pltpu
from jax.experimental.pallas import tpu_sc as plsc

_CHUNK = 64
_NCORES = 2
_NSUB = 16
_NW = _NCORES * _NSUB
_DEGW = 16
_IDXG = 8
_HH = 64


def _plan(n, n_streams):
    n_pad = ((n + 8 * _NSUB - 1) // (8 * _NSUB)) * (8 * _NSUB)
    rows_per_tile = n_pad // _NSUB
    cmax = ((-(-n_streams // _NW) + 7) // 8) * 8
    q64, rem = divmod(rows_per_tile, _CHUNK)
    return n_pad, rows_per_tile, cmax, q64, rem


def _seg_sum_half(h_half, src2d, dst2d, n_streams):
    n = h_half.shape[0]
    n_pad, rows_per_tile, cmax, q64, rem = _plan(n, n_streams)
    zrow = jnp.zeros((_CHUNK, _HH), jnp.float32)

    mesh = plsc.VectorSubcoreMesh(core_axis_name="c", subcore_axis_name="s")

    @functools.partial(
        pl.kernel,
        mesh=mesh,
        out_type=jax.ShapeDtypeStruct((_NCORES, n_pad, _HH), jnp.float32),
        scratch_types=[
            pltpu.VMEM((2, _IDXG, _CHUNK), jnp.int32),
            pltpu.VMEM((2, _IDXG, _CHUNK), jnp.int32),
            pltpu.VMEM((2, _CHUNK, _HH), jnp.float32),
            pltpu.VMEM_SHARED((n_pad, _HH), jnp.float32),
            pltpu.SemaphoreType.DMA,
            pltpu.SemaphoreType.DMA,
            pltpu.SemaphoreType.DMA,
            pltpu.SemaphoreType.DMA,
        ],
        compiler_params=pltpu.CompilerParams(use_tc_tiling_on_sc=False),
    )
    def seg_kernel(h_hbm, src_hbm, dst_hbm, zrow_hbm, parts_hbm,
                   srcg, dstg, rows, acc, g0, g1, s0, s1):
        cid = lax.axis_index("c")
        sid = lax.axis_index("s")
        w = cid * _NSUB + sid
        base = w * cmax
        cnt = jnp.clip(n_streams - base, 0, cmax)
        row0 = sid * rows_per_tile

        pltpu.sync_copy(zrow_hbm, rows.at[0])
        for j in range(q64):
            pltpu.sync_copy(rows.at[0],
                            acc.at[pl.ds(row0 + j * _CHUNK, _CHUNK)])
        if rem:
            pltpu.sync_copy(rows.at[0, pl.ds(0, rem)],
                            acc.at[pl.ds(row0 + q64 * _CHUNK, rem)])
        plsc.subcore_barrier()

        def load_idx_group(g):
            slot = g & 1
            pltpu.sync_copy(src_hbm.at[pl.ds(base + g * _IDXG, _IDXG)],
                            srcg.at[slot])
            pltpu.sync_copy(dst_hbm.at[pl.ds(base + g * _IDXG, _IDXG)],
                            dstg.at[slot])

        def src_at(k):
            return srcg.at[(k // _IDXG) & 1, k % _IDXG]

        def dst_at(k):
            return dstg.at[(k // _IDXG) & 1, k % _IDXG]

        def half_iter(k, b, gsem, gsem_n, ssem, ssem_n):
            pltpu.make_async_copy(h_hbm.at[src_at(k)], rows.at[b],
                                  gsem).wait()

            @pl.when(k + 1 < cnt)
            def _():
                @pl.when(k >= 1)
                def _():
                    pltpu.make_async_copy(
                        rows.at[1 - b], acc.at[dst_at(k - 1)], ssem_n).wait()

                pltpu.async_copy(h_hbm.at[src_at(k + 1)], rows.at[1 - b],
                                 gsem_n)

            pltpu.async_copy(rows.at[b], acc.at[dst_at(k)], ssem, add=True)

        def body(k, carry):
            @pl.when((((k + 1) % _IDXG) == 0) & (k + 1 < cnt))
            def _():
                load_idx_group((k + 1) // _IDXG)

            @pl.when((k & 1) == 0)
            def _():
                half_iter(k, 0, g0, g1, s0, s1)

            @pl.when((k & 1) == 1)
            def _():
                half_iter(k, 1, g1, g0, s1, s0)

            return carry

        def drain_scatter(k, b, ssem):
            pltpu.make_async_copy(rows.at[b], acc.at[dst_at(k)], ssem).wait()

        @pl.when(cnt > 0)
        def _():
            load_idx_group(0)
            pltpu.async_copy(h_hbm.at[src_at(0)], rows.at[0], g0)
            lax.fori_loop(0, cnt, body, 0)

            last = cnt - 1

            @pl.when((last & 1) == 0)
            def _():
                drain_scatter(last, 0, s0)

                @pl.when(cnt >= 2)
                def _():
                    drain_scatter(last - 1, 1, s1)

            @pl.when((last & 1) == 1)
            def _():
                drain_scatter(last, 1, s1)

                @pl.when(cnt >= 2)
                def _():
                    drain_scatter(last - 1, 0, s0)

        plsc.subcore_barrier()
        for j in range(q64):
            pltpu.sync_copy(acc.at[pl.ds(row0 + j * _CHUNK, _CHUNK)],
                            rows.at[0])
            pltpu.sync_copy(rows.at[0],
                            parts_hbm.at[cid, pl.ds(row0 + j * _CHUNK,
                                                    _CHUNK)])
        if rem:
            pltpu.sync_copy(acc.at[pl.ds(row0 + q64 * _CHUNK, rem)],
                            rows.at[0, pl.ds(0, rem)])
            pltpu.sync_copy(rows.at[0, pl.ds(0, rem)],
                            parts_hbm.at[cid, pl.ds(row0 + q64 * _CHUNK,
                                                    rem)])

    return seg_kernel(h_half, src2d, dst2d, zrow)


def _deg_sc(dst2d, n, n_streams):
    n_pad, rows_per_tile, cmax, q64, rem = _plan(n, n_streams)
    zdeg = jnp.zeros((_CHUNK, _DEGW), jnp.float32)
    ones16 = jnp.ones((_CHUNK, _DEGW), jnp.float32)

    mesh = plsc.VectorSubcoreMesh(core_axis_name="c", subcore_axis_name="s")

    @functools.partial(
        pl.kernel,
        mesh=mesh,
        out_type=jax.ShapeDtypeStruct((_NCORES, n_pad, _DEGW), jnp.float32),
        scratch_types=[
            pltpu.VMEM((_IDXG, _CHUNK), jnp.int32),
            pltpu.VMEM((_CHUNK, _DEGW), jnp.float32),
            pltpu.VMEM((_CHUNK, _DEGW), jnp.float32),
            pltpu.VMEM_SHARED((n_pad, _DEGW), jnp.float32),
        ],
        compiler_params=pltpu.CompilerParams(use_tc_tiling_on_sc=False),
    )
    def deg_kernel(dst_hbm, zdeg_hbm, ones_hbm, degp_hbm,
                   dstg, ones_v, deg_v, deg_sh):
        cid = lax.axis_index("c")
        sid = lax.axis_index("s")
        w = cid * _NSUB + sid
        base = w * cmax
        cnt = jnp.clip(n_streams - base, 0, cmax)
        row0 = sid * rows_per_tile

        pltpu.sync_copy(zdeg_hbm, deg_v)
        for j in range(q64):
            pltpu.sync_copy(deg_v,
                            deg_sh.at[pl.ds(row0 + j * _CHUNK, _CHUNK)])
        if rem:
            pltpu.sync_copy(deg_v.at[pl.ds(0, rem)],
                            deg_sh.at[pl.ds(row0 + q64 * _CHUNK, rem)])
        pltpu.sync_copy(ones_hbm, ones_v)
        plsc.subcore_barrier()

        def body(k, carry):
            @pl.when((k % _IDXG) == 0)
            def _():
                g = k // _IDXG
                pltpu.sync_copy(
                    dst_hbm.at[pl.ds(base + g * _IDXG, _IDXG)], dstg)

            pltpu.sync_copy(ones_v, deg_sh.at[dstg.at[k % _IDXG]], add=True)
            return carry

        lax.fori_loop(0, cnt, body, 0)

        plsc.subcore_barrier()
        for j in range(q64):
            pltpu.sync_copy(deg_sh.at[pl.ds(row0 + j * _CHUNK, _CHUNK)],
                            deg_v)
            pltpu.sync_copy(deg_v,
                            degp_hbm.at[cid, pl.ds(row0 + j * _CHUNK,
                                                   _CHUNK)])
        if rem:
            pltpu.sync_copy(deg_sh.at[pl.ds(row0 + q64 * _CHUNK, rem)],
                            deg_v.at[pl.ds(0, rem)])
            pltpu.sync_copy(deg_v.at[pl.ds(0, rem)],
                            degp_hbm.at[cid, pl.ds(row0 + q64 * _CHUNK,
                                                   rem)])

    return deg_kernel(dst2d, zdeg, ones16)


def _enc_body(x_ref, w1_ref, b1_ref, w2_ref, b2_ref, lo_ref, hi_ref):
    a = jnp.dot(x_ref[...], w1_ref[...], preferred_element_type=jnp.float32)
    a = jnp.maximum(a + b1_ref[...], 0.0)
    b = jnp.dot(a, w2_ref[...], preferred_element_type=jnp.float32)
    h = jnp.maximum(b + b2_ref[...], 0.0)
    lo_ref[...] = h[:, :_HH]
    hi_ref[...] = h[:, _HH:]


def _dec_body(plo_ref, phi_ref, degp_ref, hlo_ref, hhi_ref,
              mpwl_ref, mpwh_ref, mpb_ref, w1a_ref, w1bl_ref, w1bh_ref,
              b1_ref, w2_ref, b2_ref, o_ref):
    seg_lo = plo_ref[0] + plo_ref[1]
    seg_hi = phi_ref[0] + phi_ref[1]
    deg = degp_ref[0, :, 0:1] + degp_ref[1, :, 0:1]
    agg = (jnp.dot(seg_lo, mpwl_ref[...], preferred_element_type=jnp.float32)
           + jnp.dot(seg_hi, mpwh_ref[...],
                     preferred_element_type=jnp.float32)
           + deg * mpb_ref[...])
    u = (jnp.dot(agg, w1a_ref[...], preferred_element_type=jnp.float32)
         + jnp.dot(hlo_ref[...], w1bl_ref[...],
                   preferred_element_type=jnp.float32)
         + jnp.dot(hhi_ref[...], w1bh_ref[...],
                   preferred_element_type=jnp.float32)
         + b1_ref[...])
    u = jnp.maximum(u, 0.0)
    v = jnp.dot(u, w2_ref[...], preferred_element_type=jnp.float32)
    o_ref[...] = jnp.tanh(v + b2_ref[...])


def kernel(x, edge_index, steps, enc_W1, enc_b1, enc_W2, enc_b2, mp_W, mp_b,
           dec_W1, dec_b1, dec_W2, dec_b2):
    n, d_in = x.shape
    h_dim = mp_W.shape[0]
    mlp_h = enc_W1.shape[1]
    e = edge_index.shape[1]

    n_streams = e // _CHUNK
    cmax = ((-(-n_streams // _NW) + 7) // 8) * 8
    pad_streams = _NW * cmax

    src = edge_index[0]
    dst = edge_index[1]
    pad_e = pad_streams * _CHUNK - e
    src2d = jnp.pad(src, (0, pad_e)).reshape(pad_streams, _CHUNK)
    dst2d = jnp.pad(dst, (0, pad_e)).reshape(pad_streams, _CHUNK)

    b1e = enc_b1.reshape(1, mlp_h)
    b2e = enc_b2.reshape(1, h_dim)
    mpb = mp_b.reshape(1, h_dim)
    mpw_lo = mp_W[:_HH]
    mpw_hi = mp_W[_HH:]
    w1a = dec_W1[:h_dim]
    w1b_lo = dec_W1[h_dim:h_dim + _HH]
    w1b_hi = dec_W1[h_dim + _HH:]
    b1d = dec_b1.reshape(1, mlp_h)
    b2d = dec_b2.reshape(1, d_in)

    rb = 2000
    grid = (n // rb,)

    def full(shape):
        return pl.BlockSpec(shape, lambda i: tuple(0 for _ in shape))

    enc_call = pl.pallas_call(
        _enc_body,
        grid=grid,
        in_specs=[
            pl.BlockSpec((rb, d_in), lambda i: (i, 0)),
            full((d_in, mlp_h)),
            full((1, mlp_h)),
            full((mlp_h, h_dim)),
            full((1, h_dim)),
        ],
        out_specs=[pl.BlockSpec((rb, _HH), lambda i: (i, 0)),
                   pl.BlockSpec((rb, _HH), lambda i: (i, 0))],
        out_shape=[jax.ShapeDtypeStruct((n, _HH), jnp.float32),
                   jax.ShapeDtypeStruct((n, _HH), jnp.float32)],
    )

    dec_call = pl.pallas_call(
        _dec_body,
        grid=grid,
        in_specs=[
            pl.BlockSpec((_NCORES, rb, _HH), lambda i: (0, i, 0)),
            pl.BlockSpec((_NCORES, rb, _HH), lambda i: (0, i, 0)),
            pl.BlockSpec((_NCORES, rb, _DEGW), lambda i: (0, i, 0)),
            pl.BlockSpec((rb, _HH), lambda i: (i, 0)),
            pl.BlockSpec((rb, _HH), lambda i: (i, 0)),
            full((_HH, h_dim)),
            full((_HH, h_dim)),
            full((1, h_dim)),
            full((h_dim, mlp_h)),
            full((_HH, mlp_h)),
            full((_HH, mlp_h)),
            full((1, mlp_h)),
            full((mlp_h, d_in)),
            full((1, d_in)),
        ],
        out_specs=pl.BlockSpec((rb, d_in), lambda i: (i, 0)),
        out_shape=jax.ShapeDtypeStruct((n, d_in), jnp.float32),
    )

    degp = _deg_sc(dst2d, n, n_streams)[:, :n]

    def body(_, xc):
        h_lo, h_hi = enc_call(xc, enc_W1, b1e, enc_W2, b2e)
        p_lo = _seg_sum_half(h_lo, src2d, dst2d, n_streams)[:, :n]
        p_hi = _seg_sum_half(h_hi, src2d, dst2d, n_streams)[:, :n]
        return dec_call(p_lo, p_hi, degp, h_lo, h_hi,
                        mpw_lo, mpw_hi, mpb, w1a, w1b_lo, w1b_hi,
                        b1d, dec_W2, b2d)

    return lax.fori_loop(0, steps, body, x)


# speedup vs baseline: 6.5626x; 1.3582x over previous
import functools

import jax
import jax.numpy as jnp
from jax import lax
from jax.experimental import pallas as pl
from jax.experimental.pallas import tpu as pltpu
from jax.experimental.pallas import tpu_sc as plsc

_CHUNK = 128
_NCORES = 2
_NSUB = 16
_NW = _NCORES * _NSUB
_DEGW = 16
_IDXG = 4
_HH = 64


def _plan(n, n_streams):
    n_pad = ((n + 8 * _NSUB - 1) // (8 * _NSUB)) * (8 * _NSUB)
    rows_per_tile = n_pad // _NSUB
    cmax = ((-(-n_streams // _NW) + 7) // 8) * 8
    q64, rem = divmod(rows_per_tile, _CHUNK)
    return n_pad, rows_per_tile, cmax, q64, rem


def _seg_sum_half(h_half, src2d, dst2d, n_streams):
    n = h_half.shape[0]
    n_pad, rows_per_tile, cmax, q64, rem = _plan(n, n_streams)
    zrow = jnp.zeros((_CHUNK, _HH), jnp.float32)

    mesh = plsc.VectorSubcoreMesh(core_axis_name="c", subcore_axis_name="s")

    @functools.partial(
        pl.kernel,
        mesh=mesh,
        out_type=jax.ShapeDtypeStruct((_NCORES, n_pad, _HH), jnp.float32),
        scratch_types=[
            pltpu.VMEM((2, _IDXG, _CHUNK), jnp.int32),
            pltpu.VMEM((2, _IDXG, _CHUNK), jnp.int32),
            pltpu.VMEM((2, _CHUNK, _HH), jnp.float32),
            pltpu.VMEM_SHARED((n_pad, _HH), jnp.float32),
            pltpu.SemaphoreType.DMA,
            pltpu.SemaphoreType.DMA,
            pltpu.SemaphoreType.DMA,
            pltpu.SemaphoreType.DMA,
        ],
        compiler_params=pltpu.CompilerParams(use_tc_tiling_on_sc=False),
    )
    def seg_kernel(h_hbm, src_hbm, dst_hbm, zrow_hbm, parts_hbm,
                   srcg, dstg, rows, acc, g0, g1, s0, s1):
        cid = lax.axis_index("c")
        sid = lax.axis_index("s")
        w = cid * _NSUB + sid
        base = w * cmax
        cnt = jnp.clip(n_streams - base, 0, cmax)
        row0 = sid * rows_per_tile

        pltpu.sync_copy(zrow_hbm, rows.at[0])
        for j in range(q64):
            pltpu.sync_copy(rows.at[0],
                            acc.at[pl.ds(row0 + j * _CHUNK, _CHUNK)])
        if rem:
            pltpu.sync_copy(rows.at[0, pl.ds(0, rem)],
                            acc.at[pl.ds(row0 + q64 * _CHUNK, rem)])
        plsc.subcore_barrier()

        def load_idx_group(g):
            slot = g & 1
            pltpu.sync_copy(src_hbm.at[pl.ds(base + g * _IDXG, _IDXG)],
                            srcg.at[slot])
            pltpu.sync_copy(dst_hbm.at[pl.ds(base + g * _IDXG, _IDXG)],
                            dstg.at[slot])

        def src_at(k):
            return srcg.at[(k // _IDXG) & 1, k % _IDXG]

        def dst_at(k):
            return dstg.at[(k // _IDXG) & 1, k % _IDXG]

        def half_iter(k, b, gsem, gsem_n, ssem, ssem_n):
            pltpu.make_async_copy(h_hbm.at[src_at(k)], rows.at[b],
                                  gsem).wait()

            @pl.when(k + 1 < cnt)
            def _():
                @pl.when(k >= 1)
                def _():
                    pltpu.make_async_copy(
                        rows.at[1 - b], acc.at[dst_at(k - 1)], ssem_n).wait()

                pltpu.async_copy(h_hbm.at[src_at(k + 1)], rows.at[1 - b],
                                 gsem_n)

            pltpu.async_copy(rows.at[b], acc.at[dst_at(k)], ssem, add=True)

        def body(k, carry):
            @pl.when((((k + 1) % _IDXG) == 0) & (k + 1 < cnt))
            def _():
                load_idx_group((k + 1) // _IDXG)

            @pl.when((k & 1) == 0)
            def _():
                half_iter(k, 0, g0, g1, s0, s1)

            @pl.when((k & 1) == 1)
            def _():
                half_iter(k, 1, g1, g0, s1, s0)

            return carry

        def drain_scatter(k, b, ssem):
            pltpu.make_async_copy(rows.at[b], acc.at[dst_at(k)], ssem).wait()

        @pl.when(cnt > 0)
        def _():
            load_idx_group(0)
            pltpu.async_copy(h_hbm.at[src_at(0)], rows.at[0], g0)
            lax.fori_loop(0, cnt, body, 0)

            last = cnt - 1

            @pl.when((last & 1) == 0)
            def _():
                drain_scatter(last, 0, s0)

                @pl.when(cnt >= 2)
                def _():
                    drain_scatter(last - 1, 1, s1)

            @pl.when((last & 1) == 1)
            def _():
                drain_scatter(last, 1, s1)

                @pl.when(cnt >= 2)
                def _():
                    drain_scatter(last - 1, 0, s0)

        plsc.subcore_barrier()
        for j in range(q64):
            pltpu.sync_copy(acc.at[pl.ds(row0 + j * _CHUNK, _CHUNK)],
                            rows.at[0])
            pltpu.sync_copy(rows.at[0],
                            parts_hbm.at[cid, pl.ds(row0 + j * _CHUNK,
                                                    _CHUNK)])
        if rem:
            pltpu.sync_copy(acc.at[pl.ds(row0 + q64 * _CHUNK, rem)],
                            rows.at[0, pl.ds(0, rem)])
            pltpu.sync_copy(rows.at[0, pl.ds(0, rem)],
                            parts_hbm.at[cid, pl.ds(row0 + q64 * _CHUNK,
                                                    rem)])

    return seg_kernel(h_half, src2d, dst2d, zrow)


def _deg_sc(dst2d, n, n_streams):
    n_pad, rows_per_tile, cmax, q64, rem = _plan(n, n_streams)
    zdeg = jnp.zeros((_CHUNK, _DEGW), jnp.float32)
    ones16 = jnp.ones((_CHUNK, _DEGW), jnp.float32)

    mesh = plsc.VectorSubcoreMesh(core_axis_name="c", subcore_axis_name="s")

    @functools.partial(
        pl.kernel,
        mesh=mesh,
        out_type=jax.ShapeDtypeStruct((_NCORES, n_pad, _DEGW), jnp.float32),
        scratch_types=[
            pltpu.VMEM((_IDXG, _CHUNK), jnp.int32),
            pltpu.VMEM((_CHUNK, _DEGW), jnp.float32),
            pltpu.VMEM((_CHUNK, _DEGW), jnp.float32),
            pltpu.VMEM_SHARED((n_pad, _DEGW), jnp.float32),
        ],
        compiler_params=pltpu.CompilerParams(use_tc_tiling_on_sc=False),
    )
    def deg_kernel(dst_hbm, zdeg_hbm, ones_hbm, degp_hbm,
                   dstg, ones_v, deg_v, deg_sh):
        cid = lax.axis_index("c")
        sid = lax.axis_index("s")
        w = cid * _NSUB + sid
        base = w * cmax
        cnt = jnp.clip(n_streams - base, 0, cmax)
        row0 = sid * rows_per_tile

        pltpu.sync_copy(zdeg_hbm, deg_v)
        for j in range(q64):
            pltpu.sync_copy(deg_v,
                            deg_sh.at[pl.ds(row0 + j * _CHUNK, _CHUNK)])
        if rem:
            pltpu.sync_copy(deg_v.at[pl.ds(0, rem)],
                            deg_sh.at[pl.ds(row0 + q64 * _CHUNK, rem)])
        pltpu.sync_copy(ones_hbm, ones_v)
        plsc.subcore_barrier()

        def body(k, carry):
            @pl.when((k % _IDXG) == 0)
            def _():
                g = k // _IDXG
                pltpu.sync_copy(
                    dst_hbm.at[pl.ds(base + g * _IDXG, _IDXG)], dstg)

            pltpu.sync_copy(ones_v, deg_sh.at[dstg.at[k % _IDXG]], add=True)
            return carry

        lax.fori_loop(0, cnt, body, 0)

        plsc.subcore_barrier()
        for j in range(q64):
            pltpu.sync_copy(deg_sh.at[pl.ds(row0 + j * _CHUNK, _CHUNK)],
                            deg_v)
            pltpu.sync_copy(deg_v,
                            degp_hbm.at[cid, pl.ds(row0 + j * _CHUNK,
                                                   _CHUNK)])
        if rem:
            pltpu.sync_copy(deg_sh.at[pl.ds(row0 + q64 * _CHUNK, rem)],
                            deg_v.at[pl.ds(0, rem)])
            pltpu.sync_copy(deg_v.at[pl.ds(0, rem)],
                            degp_hbm.at[cid, pl.ds(row0 + q64 * _CHUNK,
                                                   rem)])

    return deg_kernel(dst2d, zdeg, ones16)


def _enc_body(x_ref, w1_ref, b1_ref, w2_ref, b2_ref, lo_ref, hi_ref):
    a = jnp.dot(x_ref[...], w1_ref[...], preferred_element_type=jnp.float32)
    a = jnp.maximum(a + b1_ref[...], 0.0)
    b = jnp.dot(a, w2_ref[...], preferred_element_type=jnp.float32)
    h = jnp.maximum(b + b2_ref[...], 0.0)
    lo_ref[...] = h[:, :_HH]
    hi_ref[...] = h[:, _HH:]


def _dec_body(plo_ref, phi_ref, degp_ref, hlo_ref, hhi_ref,
              mpwl_ref, mpwh_ref, mpb_ref, w1a_ref, w1bl_ref, w1bh_ref,
              b1_ref, w2_ref, b2_ref, o_ref):
    seg_lo = plo_ref[0] + plo_ref[1]
    seg_hi = phi_ref[0] + phi_ref[1]
    deg = degp_ref[0, :, 0:1] + degp_ref[1, :, 0:1]
    agg = (jnp.dot(seg_lo, mpwl_ref[...], preferred_element_type=jnp.float32)
           + jnp.dot(seg_hi, mpwh_ref[...],
                     preferred_element_type=jnp.float32)
           + deg * mpb_ref[...])
    u = (jnp.dot(agg, w1a_ref[...], preferred_element_type=jnp.float32)
         + jnp.dot(hlo_ref[...], w1bl_ref[...],
                   preferred_element_type=jnp.float32)
         + jnp.dot(hhi_ref[...], w1bh_ref[...],
                   preferred_element_type=jnp.float32)
         + b1_ref[...])
    u = jnp.maximum(u, 0.0)
    v = jnp.dot(u, w2_ref[...], preferred_element_type=jnp.float32)
    o_ref[...] = jnp.tanh(v + b2_ref[...])


def kernel(x, edge_index, steps, enc_W1, enc_b1, enc_W2, enc_b2, mp_W, mp_b,
           dec_W1, dec_b1, dec_W2, dec_b2):
    n, d_in = x.shape
    h_dim = mp_W.shape[0]
    mlp_h = enc_W1.shape[1]
    e = edge_index.shape[1]

    n_streams = e // _CHUNK
    cmax = ((-(-n_streams // _NW) + 7) // 8) * 8
    pad_streams = _NW * cmax

    src = edge_index[0]
    dst = edge_index[1]
    pad_e = pad_streams * _CHUNK - e
    src2d = jnp.pad(src, (0, pad_e)).reshape(pad_streams, _CHUNK)
    dst2d = jnp.pad(dst, (0, pad_e)).reshape(pad_streams, _CHUNK)

    b1e = enc_b1.reshape(1, mlp_h)
    b2e = enc_b2.reshape(1, h_dim)
    mpb = mp_b.reshape(1, h_dim)
    mpw_lo = mp_W[:_HH]
    mpw_hi = mp_W[_HH:]
    w1a = dec_W1[:h_dim]
    w1b_lo = dec_W1[h_dim:h_dim + _HH]
    w1b_hi = dec_W1[h_dim + _HH:]
    b1d = dec_b1.reshape(1, mlp_h)
    b2d = dec_b2.reshape(1, d_in)

    rb = 2000
    grid = (n // rb,)

    def full(shape):
        return pl.BlockSpec(shape, lambda i: tuple(0 for _ in shape))

    enc_call = pl.pallas_call(
        _enc_body,
        grid=grid,
        in_specs=[
            pl.BlockSpec((rb, d_in), lambda i: (i, 0)),
            full((d_in, mlp_h)),
            full((1, mlp_h)),
            full((mlp_h, h_dim)),
            full((1, h_dim)),
        ],
        out_specs=[pl.BlockSpec((rb, _HH), lambda i: (i, 0)),
                   pl.BlockSpec((rb, _HH), lambda i: (i, 0))],
        out_shape=[jax.ShapeDtypeStruct((n, _HH), jnp.float32),
                   jax.ShapeDtypeStruct((n, _HH), jnp.float32)],
    )

    dec_call = pl.pallas_call(
        _dec_body,
        grid=grid,
        in_specs=[
            pl.BlockSpec((_NCORES, rb, _HH), lambda i: (0, i, 0)),
            pl.BlockSpec((_NCORES, rb, _HH), lambda i: (0, i, 0)),
            pl.BlockSpec((_NCORES, rb, _DEGW), lambda i: (0, i, 0)),
            pl.BlockSpec((rb, _HH), lambda i: (i, 0)),
            pl.BlockSpec((rb, _HH), lambda i: (i, 0)),
            full((_HH, h_dim)),
            full((_HH, h_dim)),
            full((1, h_dim)),
            full((h_dim, mlp_h)),
            full((_HH, mlp_h)),
            full((_HH, mlp_h)),
            full((1, mlp_h)),
            full((mlp_h, d_in)),
            full((1, d_in)),
        ],
        out_specs=pl.BlockSpec((rb, d_in), lambda i: (i, 0)),
        out_shape=jax.ShapeDtypeStruct((n, d_in), jnp.float32),
    )

    degp = _deg_sc(dst2d, n, n_streams)[:, :n]

    def body(_, xc):
        h_lo, h_hi = enc_call(xc, enc_W1, b1e, enc_W2, b2e)
        p_lo = _seg_sum_half(h_lo, src2d, dst2d, n_streams)[:, :n]
        p_hi = _seg_sum_half(h_hi, src2d, dst2d, n_streams)[:, :n]
        return dec_call(p_lo, p_hi, degp, h_lo, h_hi,
                        mpw_lo, mpw_hi, mpb, w1a, w1b_lo, w1b_hi,
                        b1d, dec_W2, b2d)

    return lax.fori_loop(0, steps, body, x)
